# baseline (speedup 1.0000x reference)
import jax
import jax.numpy as jnp
from jax import lax
from jax.experimental import pallas as pl
from jax.experimental.pallas import tpu as pltpu
from jax.experimental.pallas import tpu_sc as plsc

N = 10000
E = 320000
F = 128
NPAD = 8
W = 256
K = W // 128
NTILES = 16
E_PAD = 327680
NW = E_PAD // (W * NTILES)
DEGR = 10240
DCH = 2048

NHALF = N // 2
AGG_ROWS = 8192
TRASH = NHALF
TRASH_MOD = 3072

_mesh = lambda: plsc.VectorSubcoreMesh(
    core_axis_name="c", subcore_axis_name="s", num_cores=2, num_subcores=16)


def _deg_body(c2_hbm, deg_hbm, deg_sh, idx_v, ones_v, deg_v):
    c = lax.axis_index("c")
    s = lax.axis_index("s")

    def zv(i, carry):
        deg_v[pl.ds(i * 16, 16)] = jnp.zeros((16,), jnp.float32)
        return carry

    lax.fori_loop(0, DCH // 16, zv, 0)

    @pl.when(s == 0)
    def _zero():
        for i in range(DEGR // DCH):
            pltpu.sync_copy(deg_v, deg_sh.at[pl.ds(i * DCH, DCH)])

    for i in range(8):
        ones_v[pl.ds(i * 16, 16)] = jnp.full((16,), 1.0, jnp.float32)
    plsc.subcore_barrier()

    def body(k, carry):
        w = s * NW + k
        pltpu.sync_copy(c2_hbm.at[pl.ds(w * K, K)], idx_v)
        for j in range(K):
            pltpu.sync_copy(ones_v, deg_sh.at[idx_v.at[j]], add=True)
        return carry

    lax.fori_loop(0, NW, body, 0)
    plsc.subcore_barrier()

    @pl.when((s == 0) & (c == 0))
    def _out():
        for i in range(DEGR // DCH):
            pltpu.sync_copy(deg_sh.at[pl.ds(i * DCH, DCH)], deg_v)
            pltpu.sync_copy(deg_v, deg_hbm.at[pl.ds(i * DCH, DCH)])


def _deg_call(c2):
    return pl.kernel(
        _deg_body,
        out_type=jax.ShapeDtypeStruct((DEGR,), jnp.float32),
        mesh=_mesh(),
        scratch_types=[
            pltpu.VMEM_SHARED((DEGR,), jnp.float32),
            pltpu.VMEM((K, 128), jnp.int32),
            pltpu.VMEM((128,), jnp.float32),
            pltpu.VMEM((DCH,), jnp.float32),
        ],
    )(c2)


LFIX = 896
REGW = LFIX * NTILES
PV_LEN = 32 * REGW
PK = 8192
PBLK = 2048


def _part_body(r2_hbm, c2_hbm, pv_hbm, cnt_hbm,
               ps_sh, rbuf_v, cbuf_v, vstage_v, ostage_v, bounce_v, cnt_v):
    c = lax.axis_index("c")
    s = lax.axis_index("s")
    rid = c * NTILES + s
    base = c * NHALF
    sbase = s * REGW
    lanes = lax.iota(jnp.int32, 16)
    bvec = jnp.broadcast_to(sbase, (16,)) + lanes
    dumpv = jnp.full((16,), NTILES * REGW, jnp.int32) + lanes
    onev = jnp.full((16,), 1, jnp.int32)
    zerov = jnp.full((16,), 0, jnp.int32)

    def wbody(k, wpv):
        w = s * NW + k
        pltpu.sync_copy(r2_hbm.at[pl.ds(w * K, K)], rbuf_v)
        pltpu.sync_copy(c2_hbm.at[pl.ds(w * K, K)], cbuf_v)
        for j in range(K):
            for g in range(8):
                r16 = rbuf_v[j, pl.ds(g * 16, 16)]
                c16 = cbuf_v[j, pl.ds(g * 16, 16)]
                lc = c16 - base
                m = (lc >= 0) & (lc < NHALF)
                v = r16 * PK + lc
                wpc = jnp.minimum(wpv, LFIX - 1)
                off = jnp.where(m, wpc * 16 + bvec, dumpv)
                vstage_v[j, pl.ds(g * 16, 16)] = v
                ostage_v[j, pl.ds(g * 16, 16)] = off
                wpv = wpv + jnp.where(m, onev, zerov)
            pltpu.sync_copy(vstage_v.at[j], ps_sh.at[ostage_v.at[j]])
        return wpv

    wpv = lax.fori_loop(0, NW, wbody, zerov)
    wpf = jnp.minimum(wpv, LFIX)
    cnt_v[...] = wpf

    def cbody(i, carry):
        pltpu.sync_copy(ps_sh.at[pl.ds(sbase + i * PBLK, PBLK)], bounce_v)

        def pbody(p0, pvec):
            cur = bounce_v[pl.ds(p0 * 16, 16)]
            t = TRASH + ((pvec + lanes * 57) & 1023)
            bounce_v[pl.ds(p0 * 16, 16)] = jnp.where(pvec >= wpf, t, cur)
            return pvec + onev

        lax.fori_loop(0, PBLK // 16, pbody, jnp.broadcast_to(i * (PBLK // 16), (16,)))
        pltpu.sync_copy(bounce_v, pv_hbm.at[pl.ds(rid * REGW + i * PBLK, PBLK)])
        return carry

    lax.fori_loop(0, REGW // PBLK, cbody, 0)


def _part_call(r2, c2):
    return pl.kernel(
        _part_body,
        out_type=[
            jax.ShapeDtypeStruct((PV_LEN,), jnp.int32),
            jax.ShapeDtypeStruct((32, 16), jnp.int32),
        ],
        mesh=_mesh(),
        scratch_types=[
            pltpu.VMEM_SHARED((NTILES * REGW + 16,), jnp.int32),
            pltpu.VMEM((K, 128), jnp.int32),
            pltpu.VMEM((K, 128), jnp.int32),
            pltpu.VMEM((K, 128), jnp.int32),
            pltpu.VMEM((K, 128), jnp.int32),
            pltpu.VMEM((PBLK,), jnp.int32),
            pltpu.VMEM((16,), jnp.int32),
        ],
    )(r2, c2)


RZ = AGG_ROWS // NTILES
RO = 312
RO_TAIL = NHALF - RO * NTILES


CH = 128
NCHS = REGW // CH


def _prop_body(h_hbm, pv_hbm, out_hbm,
               agg_sh, pv_v, idxr_v, idxc_v, rows_v,
               sem_i, sem_g, sem_s):
    c = lax.axis_index("c")
    s = lax.axis_index("s")
    rid = c * NTILES + s
    rbase = rid * REGW
    nbase = c * NHALF

    def zv(i, carry):
        rows_v[0, i // 8, pl.ds((i % 8) * 16, 16)] = jnp.zeros((16,), jnp.float32)
        return carry

    lax.fori_loop(0, CH * 8, zv, 0)
    for i in range(RZ // CH):
        pltpu.sync_copy(rows_v.at[0], agg_sh.at[pl.ds(s * RZ + i * CH, CH)])
    plsc.subcore_barrier()

    def stage_idx(chunk, slot):
        pltpu.async_copy(pv_hbm.at[pl.ds(rbase + chunk * CH, CH)],
                         pv_v.at[slot], sem_i)

    def wait_idx2():
        for _ in range(2):
            pltpu.make_async_copy(pv_hbm.at[pl.ds(0, CH)], pv_v.at[0], sem_i).wait()

    def drain_scatter(slot):
        pltpu.make_async_copy(h_hbm.at[pl.ds(0, CH)], rows_v.at[slot], sem_s).wait()

    stage_idx(0, 0)
    stage_idx(1, 1)

    def body(g, carry):
        @pl.when(g > 0)
        def _drain():
            drain_scatter(0)
            drain_scatter(1)

        wait_idx2()

        for par in range(2):
            @pl.when(lax.rem(g, 2) == par)
            def _do(par=par):
                for b in range(2):
                    sl = 2 * par + b
                    ch = 2 * g + b
                    for v in range(8):
                        pk = pv_v[sl, pl.ds(v * 16, 16)]
                        idxr_v[sl, pl.ds(v * 16, 16)] = jnp.right_shift(pk, 13)
                        idxc_v[sl, pl.ds(v * 16, 16)] = jnp.bitwise_and(pk, PK - 1)
                    pltpu.async_copy(h_hbm.at[idxr_v.at[sl]], rows_v.at[b], sem_g)

                    @pl.when(ch + 2 < NCHS)
                    def _prefetch(ch=ch, sl=sl):
                        stage_idx(ch + 2, (sl + 2) % 4)

                for b in range(2):
                    sl = 2 * par + b
                    pltpu.make_async_copy(h_hbm.at[pl.ds(0, CH)], rows_v.at[b], sem_g).wait()
                    pltpu.async_copy(rows_v.at[b], agg_sh.at[idxc_v.at[sl]], sem_s, add=True)
        return carry

    lax.fori_loop(0, NCHS // 2, body, 0)
    drain_scatter(0)
    drain_scatter(1)
    plsc.subcore_barrier()

    pltpu.sync_copy(agg_sh.at[pl.ds(s * RO, RO)],
                    out_hbm.at[pl.ds(nbase + s * RO, RO)])

    @pl.when(s == 0)
    def _out_tail():
        t = RO * NTILES
        pltpu.sync_copy(agg_sh.at[pl.ds(t, RO_TAIL)],
                        out_hbm.at[pl.ds(nbase + t, RO_TAIL)])


def _prop_call(h, pv):
    return pl.kernel(
        _prop_body,
        out_type=jax.ShapeDtypeStruct((N, F), jnp.float32),
        mesh=_mesh(),
        scratch_types=[
            pltpu.VMEM_SHARED((AGG_ROWS, F), jnp.float32),
            pltpu.VMEM((4, 128), jnp.int32),
            pltpu.VMEM((4, 128), jnp.int32),
            pltpu.VMEM((4, 128), jnp.int32),
            pltpu.VMEM((2, CH, F), jnp.float32),
            pltpu.SemaphoreType.DMA,
            pltpu.SemaphoreType.DMA,
            pltpu.SemaphoreType.DMA,
        ],
    )(h, pv)


BN = 1000
GRID = N // BN


def _dis_of(deg):
    return jnp.where(deg > 0, lax.rsqrt(deg), 0.0)


def _layer_body(x_ref, agg_ref, rp_ref, deg_ref, wi_ref, wr_ref, b_ref,
                flag_ref, h_ref, r_ref):
    dis = _dis_of(deg_ref[...])
    prev = jnp.maximum(agg_ref[...] * dis + rp_ref[...], 0.0)
    o = jnp.where(flag_ref[0, 0] > 0, x_ref[...], prev)
    h_ref[...] = jnp.dot(o, wi_ref[...], preferred_element_type=jnp.float32) * dis
    r_ref[...] = jnp.dot(o, wr_ref[...], preferred_element_type=jnp.float32) + b_ref[...]


def _layer_call(x, agg, rp, deg2, wi, wr, b, flag):
    return pl.pallas_call(
        _layer_body,
        grid=(GRID,),
        in_specs=[
            pl.BlockSpec((BN, F), lambda i: (i, 0)),
            pl.BlockSpec((BN, F), lambda i: (i, 0)),
            pl.BlockSpec((BN, F), lambda i: (i, 0)),
            pl.BlockSpec((BN, 1), lambda i: (i, 0)),
            pl.BlockSpec((F, F), lambda i: (0, 0)),
            pl.BlockSpec((F, F), lambda i: (0, 0)),
            pl.BlockSpec((1, F), lambda i: (0, 0)),
            pl.BlockSpec((1, 1), lambda i: (0, 0)),
        ],
        out_specs=[
            pl.BlockSpec((BN, F), lambda i: (i, 0)),
            pl.BlockSpec((BN, F), lambda i: (i, 0)),
        ],
        out_shape=[
            jax.ShapeDtypeStruct((N, F), jnp.float32),
            jax.ShapeDtypeStruct((N, F), jnp.float32),
        ],
    )(x, agg, rp, deg2, wi, wr, b, flag)


def _final_body(agg_ref, rp_ref, deg_ref, out_ref):
    dis = _dis_of(deg_ref[...])
    out_ref[...] = jnp.maximum(agg_ref[...] * dis + rp_ref[...], 0.0)


def _final_call(agg, rp, deg2):
    return pl.pallas_call(
        _final_body,
        grid=(GRID,),
        in_specs=[
            pl.BlockSpec((BN, F), lambda i: (i, 0)),
            pl.BlockSpec((BN, F), lambda i: (i, 0)),
            pl.BlockSpec((BN, 1), lambda i: (i, 0)),
        ],
        out_specs=pl.BlockSpec((BN, F), lambda i: (i, 0)),
        out_shape=jax.ShapeDtypeStruct((N, F), jnp.float32),
    )(agg, rp, deg2)


def kernel(x, edge_index, w_init1, w_root1, b1, w_init2, w_root2, b2,
           w_init3, w_root3, b3):
    row = edge_index[0].astype(jnp.int32)
    col = edge_index[1].astype(jnp.int32)
    npad = E_PAD - E
    pad_r = jnp.arange(npad, dtype=jnp.int32) % 16
    pad_c = N + jnp.arange(npad, dtype=jnp.int32) % NPAD
    r2 = jnp.concatenate([row, pad_r]).reshape(-1, 128)
    c2 = jnp.concatenate([col, pad_c]).reshape(-1, 128)

    deg = _deg_call(c2)
    deg2 = deg[:N].reshape(N, 1)
    pv, cnt = _part_call(r2, c2)

    wi_s = jnp.stack([w_init1, w_init2, w_init3])
    wr_s = jnp.stack([w_root1, w_root2, w_root3])
    b_s = jnp.stack([b1, b2, b3]).reshape(3, 1, F)
    flag_s = jnp.array([1.0, 0.0, 0.0], jnp.float32).reshape(3, 1, 1)

    def step(carry, ws):
        agg, r = carry
        wi, wr, b, flag = ws
        h, r_next = _layer_call(x, agg, r, deg2, wi, wr, b, flag)
        agg_next = _prop_call(h, pv)
        return (agg_next, r_next), 0.0

    init = (jnp.zeros((N, F), jnp.float32), jnp.zeros((N, F), jnp.float32))
    (agg, r), _ = lax.scan(step, init, (wi_s, wr_s, b_s, flag_s))
    return _final_call(agg, r, deg2)


# speedup vs baseline: 20.5179x; 20.5179x over previous
import jax
import jax.numpy as jnp
from jax import lax
from jax.experimental import pallas as pl
from jax.experimental.pallas import tpu as pltpu
from jax.experimental.pallas import tpu_sc as plsc

N = 10000
E = 320000
F = 128
NPAD = 8
W = 256
K = W // 128
NTILES = 16
E_PAD = 327680
NW = E_PAD // (W * NTILES)
DEGR = 10240
DCH = 2048

NHALF = N // 2
AGG_ROWS = 8192
TRASH = NHALF
TRASH_MOD = 3072

_mesh = lambda: plsc.VectorSubcoreMesh(
    core_axis_name="c", subcore_axis_name="s", num_cores=2, num_subcores=16)


def _deg_body(c2_hbm, deg_hbm, deg_sh, idx_v, ones_v, deg_v):
    c = lax.axis_index("c")
    s = lax.axis_index("s")

    def zv(i, carry):
        deg_v[pl.ds(i * 16, 16)] = jnp.zeros((16,), jnp.float32)
        return carry

    lax.fori_loop(0, DCH // 16, zv, 0)

    @pl.when(s == 0)
    def _zero():
        for i in range(DEGR // DCH):
            pltpu.sync_copy(deg_v, deg_sh.at[pl.ds(i * DCH, DCH)])

    for i in range(8):
        ones_v[pl.ds(i * 16, 16)] = jnp.full((16,), 1.0, jnp.float32)
    plsc.subcore_barrier()

    def body(k, carry):
        w = s * NW + k
        pltpu.sync_copy(c2_hbm.at[pl.ds(w * K, K)], idx_v)
        for j in range(K):
            pltpu.sync_copy(ones_v, deg_sh.at[idx_v.at[j]], add=True)
        return carry

    lax.fori_loop(0, NW, body, 0)
    plsc.subcore_barrier()

    @pl.when((s == 0) & (c == 0))
    def _out():
        for i in range(DEGR // DCH):
            pltpu.sync_copy(deg_sh.at[pl.ds(i * DCH, DCH)], deg_v)
            pltpu.sync_copy(deg_v, deg_hbm.at[pl.ds(i * DCH, DCH)])


def _deg_call(c2):
    return pl.kernel(
        _deg_body,
        out_type=jax.ShapeDtypeStruct((DEGR,), jnp.float32),
        mesh=_mesh(),
        scratch_types=[
            pltpu.VMEM_SHARED((DEGR,), jnp.float32),
            pltpu.VMEM((K, 128), jnp.int32),
            pltpu.VMEM((128,), jnp.float32),
            pltpu.VMEM((DCH,), jnp.float32),
        ],
    )(c2)


LFIX = 896
REGW = LFIX * NTILES
PV_LEN = 32 * REGW
PK = 8192
PBLK = 2048


def _part_body(r2_hbm, c2_hbm, pv_hbm, cnt_hbm,
               ps_sh, rbuf_v, cbuf_v, vstage_v, ostage_v, bounce_v, cnt_v):
    c = lax.axis_index("c")
    s = lax.axis_index("s")
    rid = c * NTILES + s
    base = c * NHALF
    sbase = s * REGW
    lanes = lax.iota(jnp.int32, 16)
    bvec = jnp.broadcast_to(sbase, (16,)) + lanes
    dumpv = jnp.full((16,), NTILES * REGW, jnp.int32) + lanes
    onev = jnp.full((16,), 1, jnp.int32)
    zerov = jnp.full((16,), 0, jnp.int32)

    def wbody(k, wpv):
        w = s * NW + k
        pltpu.sync_copy(r2_hbm.at[pl.ds(w * K, K)], rbuf_v)
        pltpu.sync_copy(c2_hbm.at[pl.ds(w * K, K)], cbuf_v)
        for j in range(K):
            for g in range(8):
                r16 = rbuf_v[j, pl.ds(g * 16, 16)]
                c16 = cbuf_v[j, pl.ds(g * 16, 16)]
                lc = c16 - base
                m = (lc >= 0) & (lc < NHALF)
                v = r16 * PK + lc
                wpc = jnp.minimum(wpv, LFIX - 1)
                off = jnp.where(m, wpc * 16 + bvec, dumpv)
                vstage_v[j, pl.ds(g * 16, 16)] = v
                ostage_v[j, pl.ds(g * 16, 16)] = off
                wpv = wpv + jnp.where(m, onev, zerov)
            pltpu.sync_copy(vstage_v.at[j], ps_sh.at[ostage_v.at[j]])
        return wpv

    wpv = lax.fori_loop(0, NW, wbody, zerov)
    wpf = jnp.minimum(wpv, LFIX)
    cnt_v[...] = wpf

    def cbody(i, carry):
        pltpu.sync_copy(ps_sh.at[pl.ds(sbase + i * PBLK, PBLK)], bounce_v)

        def pbody(p0, pvec):
            cur = bounce_v[pl.ds(p0 * 16, 16)]
            t = (pvec + lanes * 64) * PK + TRASH + ((pvec + lanes * 57) & 1023)
            bounce_v[pl.ds(p0 * 16, 16)] = jnp.where(pvec >= wpf, t, cur)
            return pvec + onev

        lax.fori_loop(0, PBLK // 16, pbody, jnp.broadcast_to(i * (PBLK // 16), (16,)))
        pltpu.sync_copy(bounce_v, pv_hbm.at[pl.ds(rid * REGW + i * PBLK, PBLK)])
        return carry

    lax.fori_loop(0, REGW // PBLK, cbody, 0)


def _part_call(r2, c2):
    return pl.kernel(
        _part_body,
        out_type=[
            jax.ShapeDtypeStruct((PV_LEN,), jnp.int32),
            jax.ShapeDtypeStruct((32, 16), jnp.int32),
        ],
        mesh=_mesh(),
        scratch_types=[
            pltpu.VMEM_SHARED((NTILES * REGW + 16,), jnp.int32),
            pltpu.VMEM((K, 128), jnp.int32),
            pltpu.VMEM((K, 128), jnp.int32),
            pltpu.VMEM((K, 128), jnp.int32),
            pltpu.VMEM((K, 128), jnp.int32),
            pltpu.VMEM((PBLK,), jnp.int32),
            pltpu.VMEM((16,), jnp.int32),
        ],
    )(r2, c2)


RZ = AGG_ROWS // NTILES
RO = 312
RO_TAIL = NHALF - RO * NTILES


CH = 128
NCHS = REGW // CH


def _prop_body(h_hbm, pv_hbm, out_hbm,
               agg_sh, pv_v, idxr_v, idxc_v, rows_v,
               sem_i, sem_g, sem_s):
    c = lax.axis_index("c")
    s = lax.axis_index("s")
    rid = c * NTILES + s
    rbase = rid * REGW
    nbase = c * NHALF

    def zv(i, carry):
        rows_v[0, i // 8, pl.ds((i % 8) * 16, 16)] = jnp.zeros((16,), jnp.float32)
        return carry

    lax.fori_loop(0, CH * 8, zv, 0)
    for i in range(RZ // CH):
        pltpu.sync_copy(rows_v.at[0], agg_sh.at[pl.ds(s * RZ + i * CH, CH)])
    plsc.subcore_barrier()

    def stage_idx(chunk, slot):
        pltpu.async_copy(pv_hbm.at[pl.ds(rbase + chunk * CH, CH)],
                         pv_v.at[slot], sem_i)

    def wait_idx2():
        for _ in range(2):
            pltpu.make_async_copy(pv_hbm.at[pl.ds(0, CH)], pv_v.at[0], sem_i).wait()

    def drain_scatter(slot):
        pltpu.make_async_copy(h_hbm.at[pl.ds(0, CH)], rows_v.at[slot], sem_s).wait()

    stage_idx(0, 0)
    stage_idx(1, 1)

    def body(g, carry):
        @pl.when(g > 0)
        def _drain():
            drain_scatter(0)
            drain_scatter(1)

        wait_idx2()

        for par in range(2):
            @pl.when(lax.rem(g, 2) == par)
            def _do(par=par):
                for b in range(2):
                    sl = 2 * par + b
                    ch = 2 * g + b
                    for v in range(8):
                        pk = pv_v[sl, pl.ds(v * 16, 16)]
                        idxr_v[sl, pl.ds(v * 16, 16)] = jnp.right_shift(pk, 13)
                        idxc_v[sl, pl.ds(v * 16, 16)] = jnp.bitwise_and(pk, PK - 1)
                    pltpu.async_copy(h_hbm.at[idxr_v.at[sl]], rows_v.at[b], sem_g)

                    @pl.when(ch + 2 < NCHS)
                    def _prefetch(ch=ch, sl=sl):
                        stage_idx(ch + 2, (sl + 2) % 4)

                for b in range(2):
                    sl = 2 * par + b
                    pltpu.make_async_copy(h_hbm.at[pl.ds(0, CH)], rows_v.at[b], sem_g).wait()
                    pltpu.async_copy(rows_v.at[b], agg_sh.at[idxc_v.at[sl]], sem_s, add=True)
        return carry

    lax.fori_loop(0, NCHS // 2, body, 0)
    drain_scatter(0)
    drain_scatter(1)
    plsc.subcore_barrier()

    pltpu.sync_copy(agg_sh.at[pl.ds(s * RO, RO)],
                    out_hbm.at[pl.ds(nbase + s * RO, RO)])

    @pl.when(s == 0)
    def _out_tail():
        t = RO * NTILES
        pltpu.sync_copy(agg_sh.at[pl.ds(t, RO_TAIL)],
                        out_hbm.at[pl.ds(nbase + t, RO_TAIL)])


def _prop_call(h, pv):
    return pl.kernel(
        _prop_body,
        out_type=jax.ShapeDtypeStruct((N, F), jnp.float32),
        mesh=_mesh(),
        scratch_types=[
            pltpu.VMEM_SHARED((AGG_ROWS, F), jnp.float32),
            pltpu.VMEM((4, 128), jnp.int32),
            pltpu.VMEM((4, 128), jnp.int32),
            pltpu.VMEM((4, 128), jnp.int32),
            pltpu.VMEM((2, CH, F), jnp.float32),
            pltpu.SemaphoreType.DMA,
            pltpu.SemaphoreType.DMA,
            pltpu.SemaphoreType.DMA,
        ],
    )(h, pv)


BN = 1000
GRID = N // BN


def _dis_of(deg):
    return jnp.where(deg > 0, lax.rsqrt(deg), 0.0)


def _layer_body(x_ref, agg_ref, rp_ref, deg_ref, wi_ref, wr_ref, b_ref,
                flag_ref, h_ref, r_ref):
    dis = _dis_of(deg_ref[...])
    prev = jnp.maximum(agg_ref[...] * dis + rp_ref[...], 0.0)
    o = jnp.where(flag_ref[0, 0] > 0, x_ref[...], prev)
    h_ref[...] = jnp.dot(o, wi_ref[...], preferred_element_type=jnp.float32) * dis
    r_ref[...] = jnp.dot(o, wr_ref[...], preferred_element_type=jnp.float32) + b_ref[...]


def _layer_call(x, agg, rp, deg2, wi, wr, b, flag):
    return pl.pallas_call(
        _layer_body,
        grid=(GRID,),
        in_specs=[
            pl.BlockSpec((BN, F), lambda i: (i, 0)),
            pl.BlockSpec((BN, F), lambda i: (i, 0)),
            pl.BlockSpec((BN, F), lambda i: (i, 0)),
            pl.BlockSpec((BN, 1), lambda i: (i, 0)),
            pl.BlockSpec((F, F), lambda i: (0, 0)),
            pl.BlockSpec((F, F), lambda i: (0, 0)),
            pl.BlockSpec((1, F), lambda i: (0, 0)),
            pl.BlockSpec((1, 1), lambda i: (0, 0)),
        ],
        out_specs=[
            pl.BlockSpec((BN, F), lambda i: (i, 0)),
            pl.BlockSpec((BN, F), lambda i: (i, 0)),
        ],
        out_shape=[
            jax.ShapeDtypeStruct((N, F), jnp.float32),
            jax.ShapeDtypeStruct((N, F), jnp.float32),
        ],
    )(x, agg, rp, deg2, wi, wr, b, flag)


def _final_body(agg_ref, rp_ref, deg_ref, out_ref):
    dis = _dis_of(deg_ref[...])
    out_ref[...] = jnp.maximum(agg_ref[...] * dis + rp_ref[...], 0.0)


def _final_call(agg, rp, deg2):
    return pl.pallas_call(
        _final_body,
        grid=(GRID,),
        in_specs=[
            pl.BlockSpec((BN, F), lambda i: (i, 0)),
            pl.BlockSpec((BN, F), lambda i: (i, 0)),
            pl.BlockSpec((BN, 1), lambda i: (i, 0)),
        ],
        out_specs=pl.BlockSpec((BN, F), lambda i: (i, 0)),
        out_shape=jax.ShapeDtypeStruct((N, F), jnp.float32),
    )(agg, rp, deg2)


def kernel(x, edge_index, w_init1, w_root1, b1, w_init2, w_root2, b2,
           w_init3, w_root3, b3):
    row = edge_index[0].astype(jnp.int32)
    col = edge_index[1].astype(jnp.int32)
    npad = E_PAD - E
    pad_r = jnp.arange(npad, dtype=jnp.int32) % 16
    pad_c = N + jnp.arange(npad, dtype=jnp.int32) % NPAD
    r2 = jnp.concatenate([row, pad_r]).reshape(-1, 128)
    c2 = jnp.concatenate([col, pad_c]).reshape(-1, 128)

    deg = _deg_call(c2)
    deg2 = deg[:N].reshape(N, 1)
    pv, cnt = _part_call(r2, c2)

    wi_s = jnp.stack([w_init1, w_init2, w_init3])
    wr_s = jnp.stack([w_root1, w_root2, w_root3])
    b_s = jnp.stack([b1, b2, b3]).reshape(3, 1, F)
    flag_s = jnp.array([1.0, 0.0, 0.0], jnp.float32).reshape(3, 1, 1)

    def step(carry, ws):
        agg, r = carry
        wi, wr, b, flag = ws
        h, r_next = _layer_call(x, agg, r, deg2, wi, wr, b, flag)
        agg_next = _prop_call(h, pv)
        return (agg_next, r_next), 0.0

    init = (jnp.zeros((N, F), jnp.float32), jnp.zeros((N, F), jnp.float32))
    (agg, r), _ = lax.scan(step, init, (wi_s, wr_s, b_s, flag_s))
    return _final_call(agg, r, deg2)
